# Initial kernel scaffold; baseline (speedup 1.0000x reference)
import jax
import jax.numpy as jnp
from jax.experimental import pallas as pl


def kernel(pred, labels):
    raise NotImplementedError("write your pallas kernel here")


# speedup vs baseline: 127.2688x; 127.2688x over previous
import functools

import jax
import jax.numpy as jnp
from jax import lax
from jax.experimental import pallas as pl
from jax.experimental.pallas import tpu as pltpu
from jax.experimental.pallas import tpu_sc as plsc

_N = 8388608
_NW = 16
_C = 2048
_PER_W = _N // _NW
_L = 16

_mesh = plsc.VectorSubcoreMesh(
    core_axis_name="c", subcore_axis_name="s", num_cores=1)


@functools.partial(
    pl.kernel,
    out_type=jax.ShapeDtypeStruct((_L,), jnp.float32),
    mesh=_mesh,
    scratch_types=[
        pltpu.VMEM((_C,), jnp.float32),
        pltpu.VMEM((_C,), jnp.int32),
        pltpu.VMEM((_L,), jnp.float32),
        pltpu.VMEM((_L,), jnp.float32),
        pltpu.VMEM((2 * _L,), jnp.float32),
        pltpu.VMEM((_L,), jnp.float32),
        pltpu.VMEM((2 * _L,), jnp.float32),
        pltpu.VMEM((2 * _L,), jnp.float32),
        pltpu.VMEM((_NW * _L,), jnp.float32),
        pltpu.VMEM((_NW * _L,), jnp.float32),
        pltpu.VMEM_SHARED((_NW * _L,), jnp.float32),
        pltpu.VMEM_SHARED((_NW * _L,), jnp.float32),
        pltpu.SemaphoreType.DMA,
        pltpu.SemaphoreType.DMA,
    ],
)
def _max_lost_sc(prd_hbm, lab_hbm, out_hbm, lab_v, prd_v, amax_v, amin_v,
                 l01_v, out_v, padmax_v, padmin_v, redmax_v, redmin_v,
                 sh_max, sh_min, sem_a, sem_b):
    sid = lax.axis_index("s")
    base = sid * _PER_W

    def body(step, state):
        amax, amin = state
        start = pl.multiple_of(base + step * _C, _C)
        cp_a = pltpu.async_copy(lab_hbm.at[pl.ds(start, _C)], lab_v, sem_a)
        cp_b = pltpu.async_copy(prd_hbm.at[pl.ds(start, _C)], prd_v, sem_b)
        cp_a.wait()
        cp_b.wait()

        def inner(k, carry):
            amax, amin = carry
            w = lab_v[pl.ds(k * _L, _L)] - prd_v[pl.ds(k * _L, _L)].astype(
                jnp.float32)
            return jnp.maximum(amax, w), jnp.minimum(amin, w)

        amax, amin = lax.fori_loop(0, _C // _L, inner, (amax, amin))
        return amax, amin

    init = (jnp.full((_L,), -1.0, jnp.float32),
            jnp.full((_L,), 1.0, jnp.float32))
    amax, amin = lax.fori_loop(0, _PER_W // _C, body, init)

    amax_v[...] = amax
    amin_v[...] = amin
    pltpu.sync_copy(amax_v, sh_max.at[pl.ds(sid * _L, _L)])
    pltpu.sync_copy(amin_v, sh_min.at[pl.ds(sid * _L, _L)])
    plsc.subcore_barrier()

    @pl.when(sid == 0)
    def _():
        pltpu.sync_copy(sh_max, redmax_v)
        pltpu.sync_copy(sh_min, redmin_v)
        pltpu.sync_copy(lab_hbm.at[pl.ds(0, 2 * _L)], l01_v)

        def red(k, carry):
            gmax, gmin = carry
            return (jnp.maximum(gmax, redmax_v[pl.ds(k * _L, _L)]),
                    jnp.minimum(gmin, redmin_v[pl.ds(k * _L, _L)]))

        gmax, gmin = lax.fori_loop(
            0, _NW, red, (jnp.full((_L,), -1.0, jnp.float32),
                          jnp.full((_L,), 1.0, jnp.float32)))

        padmax_v[pl.ds(_L, _L)] = jnp.full((_L,), -1.0, jnp.float32)
        padmin_v[pl.ds(_L, _L)] = jnp.full((_L,), 1.0, jnp.float32)
        for s in (8, 4, 2, 1):
            padmax_v[pl.ds(0, _L)] = gmax
            padmin_v[pl.ds(0, _L)] = gmin
            gmax = jnp.maximum(gmax, padmax_v[pl.ds(s, _L)])
            gmin = jnp.minimum(gmin, padmin_v[pl.ds(s, _L)])

        has1 = gmax > 0.0
        has0 = gmin <= 0.0
        v0 = l01_v[pl.ds(0, _L)]
        v1 = l01_v[pl.ds(1, _L)]
        res = jnp.where(has1, jnp.where(has0, jnp.maximum(v0, v1), v1), v0)
        out_v[...] = res
        pltpu.sync_copy(out_v, out_hbm)


def kernel(pred, labels):
    return _max_lost_sc(pred, labels)[0]


# speedup vs baseline: 1792.9672x; 14.0880x over previous
import functools

import jax
import jax.numpy as jnp
from jax import lax
from jax.experimental import pallas as pl
from jax.experimental.pallas import tpu as pltpu
from jax.experimental.pallas import tpu_sc as plsc

_N = 8388608
_NW = 16
_C = 2048
_PER_W = _N // _NW
_L = 16

_mesh = plsc.VectorSubcoreMesh(
    core_axis_name="c", subcore_axis_name="s", num_cores=1)


@functools.partial(
    pl.kernel,
    out_type=jax.ShapeDtypeStruct((_L,), jnp.float32),
    mesh=_mesh,
    scratch_types=[
        pltpu.VMEM((_C,), jnp.float32),
        pltpu.VMEM((_C,), jnp.int32),
        pltpu.VMEM((_L,), jnp.float32),
        pltpu.VMEM((_L,), jnp.float32),
        pltpu.VMEM((2 * _L,), jnp.float32),
        pltpu.VMEM((_L,), jnp.float32),
        pltpu.VMEM((2 * _L,), jnp.float32),
        pltpu.VMEM((2 * _L,), jnp.float32),
        pltpu.VMEM((_NW * _L,), jnp.float32),
        pltpu.VMEM((_NW * _L,), jnp.float32),
        pltpu.VMEM_SHARED((_NW * _L,), jnp.float32),
        pltpu.VMEM_SHARED((_NW * _L,), jnp.float32),
        pltpu.SemaphoreType.DMA,
        pltpu.SemaphoreType.DMA,
        pltpu.SMEM((1,), jnp.int32),
    ],
)
def _max_lost_sc(prd_hbm, lab_hbm, out_hbm, lab_v, prd_v, amax_v, amin_v,
                 l01_v, out_v, padmax_v, padmin_v, redmax_v, redmin_v,
                 sh_max, sh_min, sem_a, sem_b, done_s):
    sid = lax.axis_index("s")
    base = sid * _PER_W

    amax_v[...] = jnp.full((_L,), -1.0, jnp.float32)
    amin_v[...] = jnp.full((_L,), 1.0, jnp.float32)
    padmax_v[pl.ds(_L, _L)] = jnp.full((_L,), -1.0, jnp.float32)
    padmin_v[pl.ds(_L, _L)] = jnp.full((_L,), 1.0, jnp.float32)
    done_s[0] = jnp.int32(0)

    def body(step, carry):
        @pl.when(done_s[0] == 0)
        def _():
            start = pl.multiple_of(base + step * _C, _C)
            cp_a = pltpu.async_copy(lab_hbm.at[pl.ds(start, _C)], lab_v, sem_a)
            cp_b = pltpu.async_copy(prd_hbm.at[pl.ds(start, _C)], prd_v, sem_b)
            cp_a.wait()
            cp_b.wait()

            def inner(k, c):
                amax, amin = c
                w = lab_v[pl.ds(k * _L, _L)] - prd_v[pl.ds(k * _L, _L)].astype(
                    jnp.float32)
                return jnp.maximum(amax, w), jnp.minimum(amin, w)

            amax, amin = lax.fori_loop(
                0, _C // _L, inner, (amax_v[...], amin_v[...]))
            amax_v[...] = amax
            amin_v[...] = amin
            rmax, rmin = amax, amin
            for s in (8, 4, 2, 1):
                padmax_v[pl.ds(0, _L)] = rmax
                padmin_v[pl.ds(0, _L)] = rmin
                rmax = jnp.maximum(rmax, padmax_v[pl.ds(s, _L)])
                rmin = jnp.minimum(rmin, padmin_v[pl.ds(s, _L)])
            settled = (rmax[0] > 0.0) & (rmin[0] <= 0.0)
            done_s[0] = settled.astype(jnp.int32)

        return carry

    lax.fori_loop(0, _PER_W // _C, body, jnp.int32(0))

    pltpu.sync_copy(amax_v, sh_max.at[pl.ds(sid * _L, _L)])
    pltpu.sync_copy(amin_v, sh_min.at[pl.ds(sid * _L, _L)])
    plsc.subcore_barrier()

    @pl.when(sid == 0)
    def _():
        pltpu.sync_copy(sh_max, redmax_v)
        pltpu.sync_copy(sh_min, redmin_v)
        pltpu.sync_copy(lab_hbm.at[pl.ds(0, 2 * _L)], l01_v)

        def red(k, carry):
            gmax, gmin = carry
            return (jnp.maximum(gmax, redmax_v[pl.ds(k * _L, _L)]),
                    jnp.minimum(gmin, redmin_v[pl.ds(k * _L, _L)]))

        gmax, gmin = lax.fori_loop(
            0, _NW, red, (jnp.full((_L,), -1.0, jnp.float32),
                          jnp.full((_L,), 1.0, jnp.float32)))

        padmax_v[pl.ds(_L, _L)] = jnp.full((_L,), -1.0, jnp.float32)
        padmin_v[pl.ds(_L, _L)] = jnp.full((_L,), 1.0, jnp.float32)
        for s in (8, 4, 2, 1):
            padmax_v[pl.ds(0, _L)] = gmax
            padmin_v[pl.ds(0, _L)] = gmin
            gmax = jnp.maximum(gmax, padmax_v[pl.ds(s, _L)])
            gmin = jnp.minimum(gmin, padmin_v[pl.ds(s, _L)])

        has1 = gmax > 0.0
        has0 = gmin <= 0.0
        v0 = l01_v[pl.ds(0, _L)]
        v1 = l01_v[pl.ds(1, _L)]
        res = jnp.where(has1, jnp.where(has0, jnp.maximum(v0, v1), v1), v0)
        out_v[...] = res
        pltpu.sync_copy(out_v, out_hbm)


def kernel(pred, labels):
    return _max_lost_sc(pred, labels)[0]


# speedup vs baseline: 2013.5469x; 1.1230x over previous
import functools

import jax
import jax.numpy as jnp
from jax import lax
from jax.experimental import pallas as pl
from jax.experimental.pallas import tpu as pltpu
from jax.experimental.pallas import tpu_sc as plsc

_N = 8388608
_NW = 16
_C = 1024
_INNER = 16
_PER_W = _N // _NW
_OUTER = _PER_W // (_C * _INNER)
_L = 16

_mesh = plsc.VectorSubcoreMesh(
    core_axis_name="c", subcore_axis_name="s", num_cores=1)


@functools.partial(
    pl.kernel,
    out_type=jax.ShapeDtypeStruct((_L,), jnp.float32),
    mesh=_mesh,
    scratch_types=[
        pltpu.VMEM((_C,), jnp.float32),
        pltpu.VMEM((_C,), jnp.int32),
        pltpu.VMEM((2 * _L,), jnp.float32),
        pltpu.VMEM((2 * _L,), jnp.float32),
        pltpu.VMEM((_L,), jnp.float32),
        pltpu.VMEM((2 * _L,), jnp.float32),
        pltpu.VMEM((2 * _L,), jnp.float32),
        pltpu.VMEM((2 * _NW * _L,), jnp.float32),
        pltpu.VMEM_SHARED((2 * _NW * _L,), jnp.float32),
        pltpu.SemaphoreType.DMA,
        pltpu.SemaphoreType.DMA,
        pltpu.SMEM((1,), jnp.int32),
    ],
)
def _max_lost_sc(prd_hbm, lab_hbm, out_hbm, lab_v, prd_v, acc_v, l01_v,
                 out_v, padmax_v, padmin_v, red_v, sh_v, sem_a, sem_b,
                 done_s):
    sid = lax.axis_index("s")
    base = sid * _PER_W

    acc_v[pl.ds(0, _L)] = jnp.full((_L,), -1.0, jnp.float32)
    acc_v[pl.ds(_L, _L)] = jnp.full((_L,), 1.0, jnp.float32)
    padmax_v[pl.ds(_L, _L)] = jnp.full((_L,), -1.0, jnp.float32)
    padmin_v[pl.ds(_L, _L)] = jnp.full((_L,), 1.0, jnp.float32)
    done_s[0] = jnp.int32(0)

    def chunk(cidx):
        start = pl.multiple_of(base + cidx * _C, _C)
        cp_a = pltpu.async_copy(lab_hbm.at[pl.ds(start, _C)], lab_v, sem_a)
        cp_b = pltpu.async_copy(prd_hbm.at[pl.ds(start, _C)], prd_v, sem_b)
        cp_a.wait()
        cp_b.wait()

        def inner(k, c):
            amax, amin = c
            w0 = lab_v[pl.ds(k * 2 * _L, _L)] - prd_v[
                pl.ds(k * 2 * _L, _L)].astype(jnp.float32)
            w1 = lab_v[pl.ds(k * 2 * _L + _L, _L)] - prd_v[
                pl.ds(k * 2 * _L + _L, _L)].astype(jnp.float32)
            return (jnp.maximum(jnp.maximum(amax, w0), w1),
                    jnp.minimum(jnp.minimum(amin, w0), w1))

        amax, amin = lax.fori_loop(
            0, _C // (2 * _L), inner,
            (acc_v[pl.ds(0, _L)], acc_v[pl.ds(_L, _L)]))
        acc_v[pl.ds(0, _L)] = amax
        acc_v[pl.ds(_L, _L)] = amin

        rmax, rmin = amax, amin
        for s in (8, 4, 2, 1):
            padmax_v[pl.ds(0, _L)] = rmax
            padmin_v[pl.ds(0, _L)] = rmin
            rmax = jnp.maximum(rmax, padmax_v[pl.ds(s, _L)])
            rmin = jnp.minimum(rmin, padmin_v[pl.ds(s, _L)])
        settled = (rmax[0] > 0.0) & (rmin[0] <= 0.0)
        done_s[0] = settled.astype(jnp.int32)

    def outer(o, carry):
        @pl.when(done_s[0] == 0)
        def _():
            def inner_chunk(j, c2):
                @pl.when(done_s[0] == 0)
                def _():
                    chunk(o * _INNER + j)
                return c2

            lax.fori_loop(0, _INNER, inner_chunk, jnp.int32(0))
        return carry

    lax.fori_loop(0, _OUTER, outer, jnp.int32(0))

    pltpu.sync_copy(acc_v, sh_v.at[pl.ds(sid * 2 * _L, 2 * _L)])
    plsc.subcore_barrier()

    @pl.when(sid == 0)
    def _():
        cp_l = pltpu.async_copy(lab_hbm.at[pl.ds(0, 2 * _L)], l01_v, sem_a)
        pltpu.sync_copy(sh_v, red_v)
        cp_l.wait()

        def red(k, carry):
            gmax, gmin = carry
            return (jnp.maximum(gmax, red_v[pl.ds(k * 2 * _L, _L)]),
                    jnp.minimum(gmin, red_v[pl.ds(k * 2 * _L + _L, _L)]))

        gmax, gmin = lax.fori_loop(
            0, _NW, red, (jnp.full((_L,), -1.0, jnp.float32),
                          jnp.full((_L,), 1.0, jnp.float32)))

        for s in (8, 4, 2, 1):
            padmax_v[pl.ds(0, _L)] = gmax
            padmin_v[pl.ds(0, _L)] = gmin
            gmax = jnp.maximum(gmax, padmax_v[pl.ds(s, _L)])
            gmin = jnp.minimum(gmin, padmin_v[pl.ds(s, _L)])

        has1 = gmax > 0.0
        has0 = gmin <= 0.0
        v0 = l01_v[pl.ds(0, _L)]
        v1 = l01_v[pl.ds(1, _L)]
        res = jnp.where(has1, jnp.where(has0, jnp.maximum(v0, v1), v1), v0)
        out_v[...] = res
        pltpu.sync_copy(out_v, out_hbm)


def kernel(pred, labels):
    return _max_lost_sc(pred, labels)[0]


# speedup vs baseline: 2050.4902x; 1.0183x over previous
import functools

import jax
import jax.numpy as jnp
from jax import lax
from jax.experimental import pallas as pl
from jax.experimental.pallas import tpu as pltpu
from jax.experimental.pallas import tpu_sc as plsc

_N = 8388608
_NW = 16
_C = 1024
_INNER = 16
_PER_W = _N // _NW
_OUTER = _PER_W // (_C * _INNER)
_L = 16

_mesh = plsc.VectorSubcoreMesh(
    core_axis_name="c", subcore_axis_name="s", num_cores=1)


@functools.partial(
    pl.kernel,
    out_type=jax.ShapeDtypeStruct((_L,), jnp.float32),
    mesh=_mesh,
    scratch_types=[
        pltpu.VMEM((_C,), jnp.float32),
        pltpu.VMEM((_C,), jnp.int32),
        pltpu.VMEM((2 * _L,), jnp.float32),
        pltpu.VMEM((2 * _L,), jnp.float32),
        pltpu.VMEM((_L,), jnp.float32),
        pltpu.VMEM((2 * _L,), jnp.float32),
        pltpu.VMEM((2 * _L,), jnp.float32),
        pltpu.VMEM((2 * _NW * _L,), jnp.float32),
        pltpu.VMEM_SHARED((2 * _NW * _L,), jnp.float32),
        pltpu.SemaphoreType.DMA,
        pltpu.SemaphoreType.DMA,
        pltpu.SemaphoreType.DMA,
        pltpu.SMEM((1,), jnp.int32),
    ],
)
def _max_lost_sc(prd_hbm, lab_hbm, out_hbm, lab_v, prd_v, acc_v, l01_v,
                 out_v, padmax_v, padmin_v, red_v, sh_v, sem_a, sem_b,
                 sem_l, done_s):
    sid = lax.axis_index("s")
    base = sid * _PER_W

    @pl.when(sid == 0)
    def _():
        pltpu.async_copy(lab_hbm.at[pl.ds(0, 2 * _L)], l01_v, sem_l)

    acc_v[pl.ds(0, _L)] = jnp.full((_L,), -1.0, jnp.float32)
    acc_v[pl.ds(_L, _L)] = jnp.full((_L,), 1.0, jnp.float32)
    padmax_v[pl.ds(_L, _L)] = jnp.full((_L,), -1.0, jnp.float32)
    padmin_v[pl.ds(_L, _L)] = jnp.full((_L,), 1.0, jnp.float32)
    done_s[0] = jnp.int32(0)

    def chunk(cidx):
        start = pl.multiple_of(base + cidx * _C, _C)
        cp_a = pltpu.async_copy(lab_hbm.at[pl.ds(start, _C)], lab_v, sem_a)
        cp_b = pltpu.async_copy(prd_hbm.at[pl.ds(start, _C)], prd_v, sem_b)
        cp_a.wait()
        cp_b.wait()

        def inner(k, c):
            amax, amin = c
            w0 = lab_v[pl.ds(k * 2 * _L, _L)] - prd_v[
                pl.ds(k * 2 * _L, _L)].astype(jnp.float32)
            w1 = lab_v[pl.ds(k * 2 * _L + _L, _L)] - prd_v[
                pl.ds(k * 2 * _L + _L, _L)].astype(jnp.float32)
            return (jnp.maximum(jnp.maximum(amax, w0), w1),
                    jnp.minimum(jnp.minimum(amin, w0), w1))

        amax, amin = lax.fori_loop(
            0, _C // (2 * _L), inner,
            (acc_v[pl.ds(0, _L)], acc_v[pl.ds(_L, _L)]))
        acc_v[pl.ds(0, _L)] = amax
        acc_v[pl.ds(_L, _L)] = amin

        rmax, rmin = amax, amin
        for s in (8, 4, 2, 1):
            padmax_v[pl.ds(0, _L)] = rmax
            padmin_v[pl.ds(0, _L)] = rmin
            rmax = jnp.maximum(rmax, padmax_v[pl.ds(s, _L)])
            rmin = jnp.minimum(rmin, padmin_v[pl.ds(s, _L)])
        settled = (rmax[0] > 0.0) & (rmin[0] <= 0.0)
        done_s[0] = settled.astype(jnp.int32)

    def outer(o, carry):
        @pl.when(done_s[0] == 0)
        def _():
            def inner_chunk(j, c2):
                @pl.when(done_s[0] == 0)
                def _():
                    chunk(o * _INNER + j)
                return c2

            lax.fori_loop(0, _INNER, inner_chunk, jnp.int32(0))
        return carry

    lax.fori_loop(0, _OUTER, outer, jnp.int32(0))

    pltpu.sync_copy(acc_v, sh_v.at[pl.ds(sid * 2 * _L, 2 * _L)])
    plsc.subcore_barrier()

    @pl.when(sid == 0)
    def _():
        pltpu.sync_copy(sh_v, red_v)
        pltpu.make_async_copy(
            lab_hbm.at[pl.ds(0, 2 * _L)], l01_v, sem_l).wait()

        def red(k, carry):
            gmax, gmin = carry
            return (jnp.maximum(gmax, red_v[pl.ds(k * 2 * _L, _L)]),
                    jnp.minimum(gmin, red_v[pl.ds(k * 2 * _L + _L, _L)]))

        gmax, gmin = lax.fori_loop(
            0, _NW, red, (jnp.full((_L,), -1.0, jnp.float32),
                          jnp.full((_L,), 1.0, jnp.float32)))

        for s in (8, 4, 2, 1):
            padmax_v[pl.ds(0, _L)] = gmax
            padmin_v[pl.ds(0, _L)] = gmin
            gmax = jnp.maximum(gmax, padmax_v[pl.ds(s, _L)])
            gmin = jnp.minimum(gmin, padmin_v[pl.ds(s, _L)])

        has1 = gmax > 0.0
        has0 = gmin <= 0.0
        v0 = l01_v[pl.ds(0, _L)]
        v1 = l01_v[pl.ds(1, _L)]
        res = jnp.where(has1, jnp.where(has0, jnp.maximum(v0, v1), v1), v0)
        out_v[...] = res
        pltpu.sync_copy(out_v, out_hbm)


def kernel(pred, labels):
    return _max_lost_sc(pred, labels)[0]
